# baseline (speedup 1.0000x reference)
import jax
import jax.numpy as jnp
from jax.experimental import pallas as pl

_F = 6600
_J = 22
_TN = 40
_F_BLK = 600


def _fwd_kernel(fr_ref, flx_ref, fly_ref, tbig_ref, e_ref,
                sig_ref, amax_ref, smax_ref, reax_ref, out_ref):
    fr = fr_ref[0]
    x = fr[:, 1:2]
    y = fr[:, 2:3]
    vx = fr[:, 3:4]
    vy = fr[:, 4:5]
    ax = fr[:, 5:6]
    ay = fr[:, 6:7]

    sigma = sig_ref[0, 0]
    a_max = amax_ref[0, 0]
    s_max = smax_ref[0, 0]
    reax_t = reax_ref[0, 0]

    vxr = ax * reax_t + vx
    vyr = ay * reax_t + vy
    xr = x + vx * reax_t + 0.5 * ax * reax_t * reax_t
    yr = y + vy * reax_t + 0.5 * ay * reax_t * reax_t

    flx = flx_ref[0]
    fly = fly_ref[0]

    dx = flx - xr
    dy = fly - yr
    dmag = jnp.sqrt(dx * dx + dy * dy)
    s0 = jnp.clip((dx * vxr + dy * vyr) / dmag, -s_max, s_max)
    t_lt = (s_max - s0) / a_max
    d_lt = t_lt * (s0 + s_max) * 0.5
    soa = s0 / a_max
    t_lt = jnp.where(d_lt > dmag,
                     -soa + jnp.sqrt(soa * soa + 2.0 * dmag / a_max),
                     t_lt)
    d_lt = jnp.maximum(jnp.minimum(d_lt, dmag), 0.0)
    t_tot = reax_t + t_lt + (dmag - d_lt) / s_max

    kk = (jnp.pi / jnp.sqrt(3.0)) / sigma
    z = kk * t_tot

    zexp = jax.lax.dot_general(
        z, e_ref[...],
        dimension_numbers=(((0,), (0,)), ((), ())),
        preferred_element_type=jnp.float32,
        precision=jax.lax.Precision.HIGHEST)

    u = kk * tbig_ref[...] - zexp
    out_ref[0] = jax.nn.sigmoid(u)


def kernel(frame, tti_sigma, a_max, s_max, reax_t):
    B = frame.shape[0]
    nf = _F // _F_BLK

    x = jnp.linspace(0.5, 119.5, 120).astype(jnp.float32)
    y = jnp.linspace(-0.5, 53.5, 55).astype(jnp.float32)
    y = y.at[0].set(-0.2)
    yy, xx = jnp.meshgrid(y, x, indexing='ij')
    flx = xx.reshape(nf, 1, _F_BLK)
    fly = yy.reshape(nf, 1, _F_BLK)

    T = jnp.linspace(0.1, 4.0, _TN).astype(jnp.float32)
    tbig = jnp.repeat(T, _J).reshape(1, _TN * _J)
    e = jnp.tile(jnp.eye(_J, dtype=jnp.float32), (1, _TN))

    def s11(v):
        return jnp.asarray(v, jnp.float32).reshape(1, 1)

    out = pl.pallas_call(
        _fwd_kernel,
        grid=(B, nf),
        in_specs=[
            pl.BlockSpec((1, _J, 12), lambda b, f: (b, 0, 0)),
            pl.BlockSpec((1, 1, _F_BLK), lambda b, f: (f, 0, 0)),
            pl.BlockSpec((1, 1, _F_BLK), lambda b, f: (f, 0, 0)),
            pl.BlockSpec((1, _TN * _J), lambda b, f: (0, 0)),
            pl.BlockSpec((_J, _TN * _J), lambda b, f: (0, 0)),
            pl.BlockSpec((1, 1), lambda b, f: (0, 0)),
            pl.BlockSpec((1, 1), lambda b, f: (0, 0)),
            pl.BlockSpec((1, 1), lambda b, f: (0, 0)),
            pl.BlockSpec((1, 1), lambda b, f: (0, 0)),
        ],
        out_specs=pl.BlockSpec((1, _F_BLK, _TN * _J), lambda b, f: (b, f, 0)),
        out_shape=jax.ShapeDtypeStruct((B, _F, _TN * _J), jnp.float32),
    )(frame, flx, fly, tbig, e,
      s11(tti_sigma), s11(a_max), s11(s_max), s11(reax_t))

    return out.reshape(B, _F, _TN, _J)


# speedup vs baseline: 1.2264x; 1.2264x over previous
import jax
import jax.numpy as jnp
from jax.experimental import pallas as pl

_F = 6600
_J = 22
_TN = 40
_F_BLK = 1320


def _fwd_kernel(fr_ref, flx_ref, fly_ref, tbig_ref, e_ref,
                sig_ref, amax_ref, smax_ref, reax_ref, out_ref):
    fr = fr_ref[0]
    x = fr[:, 1:2]
    y = fr[:, 2:3]
    vx = fr[:, 3:4]
    vy = fr[:, 4:5]
    ax = fr[:, 5:6]
    ay = fr[:, 6:7]

    sigma = sig_ref[0, 0]
    a_max = amax_ref[0, 0]
    s_max = smax_ref[0, 0]
    reax_t = reax_ref[0, 0]

    vxr = ax * reax_t + vx
    vyr = ay * reax_t + vy
    xr = x + vx * reax_t + 0.5 * ax * reax_t * reax_t
    yr = y + vy * reax_t + 0.5 * ay * reax_t * reax_t

    flx = flx_ref[0]
    fly = fly_ref[0]

    dx = flx - xr
    dy = fly - yr
    dmag = jnp.sqrt(dx * dx + dy * dy)
    s0 = jnp.clip((dx * vxr + dy * vyr) / dmag, -s_max, s_max)
    t_lt = (s_max - s0) / a_max
    d_lt = t_lt * (s0 + s_max) * 0.5
    soa = s0 / a_max
    t_lt = jnp.where(d_lt > dmag,
                     -soa + jnp.sqrt(soa * soa + 2.0 * dmag / a_max),
                     t_lt)
    d_lt = jnp.maximum(jnp.minimum(d_lt, dmag), 0.0)
    t_tot = reax_t + t_lt + (dmag - d_lt) / s_max

    kk = (jnp.pi / jnp.sqrt(3.0)) / sigma
    z = kk * t_tot

    z_hi = z.astype(jnp.bfloat16)
    z_lo = (z - z_hi.astype(jnp.float32)).astype(jnp.bfloat16)
    e = e_ref[...]
    dn = (((0,), (0,)), ((), ()))
    zexp = (jax.lax.dot_general(z_hi, e, dn,
                                preferred_element_type=jnp.float32)
            + jax.lax.dot_general(z_lo, e, dn,
                                  preferred_element_type=jnp.float32))

    u = kk * tbig_ref[...] - zexp
    out_ref[0] = jax.nn.sigmoid(u)


def kernel(frame, tti_sigma, a_max, s_max, reax_t):
    B = frame.shape[0]
    nf = _F // _F_BLK

    x = jnp.linspace(0.5, 119.5, 120).astype(jnp.float32)
    y = jnp.linspace(-0.5, 53.5, 55).astype(jnp.float32)
    y = y.at[0].set(-0.2)
    yy, xx = jnp.meshgrid(y, x, indexing='ij')
    flx = xx.reshape(nf, 1, _F_BLK)
    fly = yy.reshape(nf, 1, _F_BLK)

    T = jnp.linspace(0.1, 4.0, _TN).astype(jnp.float32)
    tbig = jnp.repeat(T, _J).reshape(1, _TN * _J)
    e = jnp.tile(jnp.eye(_J, dtype=jnp.bfloat16), (1, _TN))

    def s11(v):
        return jnp.asarray(v, jnp.float32).reshape(1, 1)

    out = pl.pallas_call(
        _fwd_kernel,
        grid=(B, nf),
        in_specs=[
            pl.BlockSpec((1, _J, 12), lambda b, f: (b, 0, 0)),
            pl.BlockSpec((1, 1, _F_BLK), lambda b, f: (f, 0, 0)),
            pl.BlockSpec((1, 1, _F_BLK), lambda b, f: (f, 0, 0)),
            pl.BlockSpec((1, _TN * _J), lambda b, f: (0, 0)),
            pl.BlockSpec((_J, _TN * _J), lambda b, f: (0, 0)),
            pl.BlockSpec((1, 1), lambda b, f: (0, 0)),
            pl.BlockSpec((1, 1), lambda b, f: (0, 0)),
            pl.BlockSpec((1, 1), lambda b, f: (0, 0)),
            pl.BlockSpec((1, 1), lambda b, f: (0, 0)),
        ],
        out_specs=pl.BlockSpec((1, _F_BLK, _TN * _J), lambda b, f: (b, f, 0)),
        out_shape=jax.ShapeDtypeStruct((B, _F, _TN * _J), jnp.float32),
    )(frame, flx, fly, tbig, e,
      s11(tti_sigma), s11(a_max), s11(s_max), s11(reax_t))

    return out.reshape(B, _F, _TN, _J)
